# baseline (speedup 1.0000x reference)
import functools

import jax
import jax.numpy as jnp
from jax import lax
from jax.experimental import pallas as pl
from jax.experimental.pallas import tpu as pltpu
from jax.experimental.pallas import tpu_sc as plsc

BATCH = 16384
NUSERS = 100000
MF = 200
DMLP = 32
PW = 128
NC = 2
NS = 16
NW = NC * NS
ROWS_PER_W = BATCH // NW
CHUNK = 128
NCHUNK = ROWS_PER_W // CHUNK
PBLK = 2048


def _tc_build_p(mut, mit):
    def body(mut_ref, mit_ref, p_ref):
        p_ref[:, 0:DMLP] = jnp.transpose(mut_ref[...])
        p_ref[:, DMLP:2 * DMLP] = jnp.transpose(mit_ref[...])

    return pl.pallas_call(
        body,
        grid=(pl.cdiv(NUSERS, PBLK),),
        in_specs=[
            pl.BlockSpec((DMLP, PBLK), lambda i: (0, i)),
            pl.BlockSpec((DMLP, PBLK), lambda i: (0, i)),
        ],
        out_specs=pl.BlockSpec((PBLK, PW), lambda i: (i, 0)),
        out_shape=jax.ShapeDtypeStruct((NUSERS, PW), jnp.float32),
    )(mut, mit)


def _tc_build_s(mf_t, wp_col):
    def body(mft_ref, wp_ref, s_ref):
        x = mft_ref[...]
        s_ref[...] = jnp.sum(x * x * wp_ref[...], axis=0)

    return pl.pallas_call(
        body,
        grid=(pl.cdiv(NUSERS, PBLK),),
        in_specs=[
            pl.BlockSpec((MF, PBLK), lambda i: (0, i)),
            pl.BlockSpec((MF, 1), lambda i: (0, 0)),
        ],
        out_specs=pl.BlockSpec((PBLK,), lambda i: (i,)),
        out_shape=jax.ShapeDtypeStruct((NUSERS,), jnp.float32),
    )(mf_t, wp_col)


def _sc_mesh_kernel(**kw):
    return functools.partial(
        pl.kernel,
        mesh=plsc.VectorSubcoreMesh(core_axis_name="c", subcore_axis_name="s"),
        compiler_params=pltpu.CompilerParams(needs_layout_passes=False),
        **kw,
    )


def _sc_gather_p(userinput, iteminput, p_table):
    @_sc_mesh_kernel(
        out_type=jax.ShapeDtypeStruct((BATCH, PW), jnp.float32),
        scratch_types=[
            pltpu.VMEM((CHUNK,), jnp.int32),
            pltpu.VMEM((CHUNK,), jnp.int32),
            pltpu.VMEM((CHUNK, PW), jnp.float32),
            pltpu.VMEM((CHUNK, PW), jnp.float32),
            pltpu.SemaphoreType.DMA,
            pltpu.SemaphoreType.DMA,
        ],
    )
    def k(u_hbm, i_hbm, p_hbm, out_cat,
          uidx, iidx, urows, irows, sem1, sem2):
        wid = lax.axis_index("s") * NC + lax.axis_index("c")
        for c in range(NCHUNK):
            rb = pl.multiple_of(wid * ROWS_PER_W + c * CHUNK, CHUNK)
            pltpu.sync_copy(u_hbm.at[pl.ds(rb, CHUNK)], uidx)
            pltpu.sync_copy(i_hbm.at[pl.ds(rb, CHUNK)], iidx)
            cp1 = pltpu.async_copy(p_hbm.at[uidx], urows, sem1)
            cp2 = pltpu.async_copy(p_hbm.at[iidx], irows, sem2)
            cp1.wait()
            cp2.wait()

            def merge(r, carry):
                urows[r, pl.ds(DMLP, 16)] = irows[r, pl.ds(DMLP, 16)]
                urows[r, pl.ds(DMLP + 16, 16)] = irows[r, pl.ds(DMLP + 16, 16)]
                return carry

            lax.fori_loop(0, CHUNK, merge, 0)
            pltpu.sync_copy(urows, out_cat.at[pl.ds(rb, CHUNK)])

    return k(userinput, iteminput, p_table)


def _sc_gather_s(userinput, s_table):
    @_sc_mesh_kernel(
        out_type=jax.ShapeDtypeStruct((BATCH,), jnp.float32),
        scratch_types=[
            pltpu.VMEM((CHUNK,), jnp.int32),
            pltpu.VMEM((CHUNK,), jnp.float32),
            pltpu.SemaphoreType.DMA,
        ],
    )
    def k(u_hbm, s_hbm, out_s, uidx, svals, sem0):
        wid = lax.axis_index("s") * NC + lax.axis_index("c")
        for c in range(NCHUNK):
            rb = pl.multiple_of(wid * ROWS_PER_W + c * CHUNK, CHUNK)
            pltpu.sync_copy(u_hbm.at[pl.ds(rb, CHUNK)], uidx)
            pltpu.async_copy(s_hbm.at[uidx], svals, sem0).wait()
            pltpu.sync_copy(svals, out_s.at[pl.ds(rb, CHUNK)])

    return k(userinput, s_table)


def _tc_head(x_cat, mf_part, w1t, b1, w2t, b2, w3t, b3, wpm, bp):
    def body(x_ref, mf_ref, w1_ref, b1_ref, w2_ref, b2_ref,
             w3_ref, b3_ref, wpm_ref, bp_ref, o_ref):
        h = jnp.dot(x_ref[:, 0:2 * DMLP], w1_ref[...],
                    preferred_element_type=jnp.float32)
        h = jnp.maximum(h + b1_ref[...], 0.0)
        h = jnp.maximum(jnp.dot(h, w2_ref[...],
                                preferred_element_type=jnp.float32)
                        + b2_ref[...], 0.0)
        h = jnp.maximum(jnp.dot(h, w3_ref[...],
                                preferred_element_type=jnp.float32)
                        + b3_ref[...], 0.0)
        logit = jnp.sum(h * wpm_ref[...], axis=-1)
        logit = logit + mf_ref[...] + bp_ref[0, 0]
        o_ref[...] = 1.0 / (1.0 + jnp.exp(-logit))

    return pl.pallas_call(
        body,
        out_shape=jax.ShapeDtypeStruct((BATCH,), jnp.float32),
    )(x_cat, mf_part, w1t, b1, w2t, b2, w3t, b3, wpm, bp)


def kernel(userinput, iteminput, mf_user_w, mf_item_w, mlp_user_w,
           mlp_item_w, W1, b1, W2, b2, W3, b3, Wp, bp):
    del mf_item_w
    wp = Wp.reshape(-1)
    p_table = _tc_build_p(mlp_user_w.T, mlp_item_w.T)
    x_cat = _sc_gather_p(userinput, iteminput, p_table)
    s_table = _tc_build_s(mf_user_w.T, wp[:MF].reshape(MF, 1))
    mf_part = _sc_gather_s(userinput, s_table)
    out = _tc_head(
        x_cat, mf_part,
        W1.T, b1.reshape(1, -1),
        W2.T, b2.reshape(1, -1), W3.T, b3.reshape(1, -1),
        wp[MF:].reshape(1, -1), bp.reshape(1, 1))
    return out


# speedup vs baseline: 1.2365x; 1.2365x over previous
import functools

import jax
import jax.numpy as jnp
from jax import lax
from jax.experimental import pallas as pl
from jax.experimental.pallas import tpu as pltpu
from jax.experimental.pallas import tpu_sc as plsc

BATCH = 16384
NUSERS = 100000
MF = 200
DMLP = 32
PW = 128
NC = 2
NS = 16
NW = NC * NS
ROWS_PER_W = BATCH // NW
CHUNK = 128
NCHUNK = ROWS_PER_W // CHUNK
PBLK = 4096


def _tc_build(mf_t, mut, mit, wp_col):
    def body(mft_ref, mut_ref, mit_ref, wp_ref, p_ref, s_ref):
        x = mft_ref[...]
        s_ref[...] = jnp.sum(x * x * wp_ref[...], axis=0)
        p_ref[:, 0:DMLP] = jnp.transpose(mut_ref[...])
        p_ref[:, DMLP:2 * DMLP] = jnp.transpose(mit_ref[...])

    return pl.pallas_call(
        body,
        grid=(pl.cdiv(NUSERS, PBLK),),
        in_specs=[
            pl.BlockSpec((MF, PBLK), lambda i: (0, i)),
            pl.BlockSpec((DMLP, PBLK), lambda i: (0, i)),
            pl.BlockSpec((DMLP, PBLK), lambda i: (0, i)),
            pl.BlockSpec((MF, 1), lambda i: (0, 0)),
        ],
        out_specs=(pl.BlockSpec((PBLK, PW), lambda i: (i, 0)),
                   pl.BlockSpec((PBLK,), lambda i: (i,))),
        out_shape=(jax.ShapeDtypeStruct((NUSERS, PW), jnp.float32),
                   jax.ShapeDtypeStruct((NUSERS,), jnp.float32)),
    )(mf_t, mut, mit, wp_col)


def _sc_gather(userinput, iteminput, p_table, s_table):
    mesh = plsc.VectorSubcoreMesh(core_axis_name="c", subcore_axis_name="s")

    @functools.partial(
        pl.kernel,
        mesh=mesh,
        compiler_params=pltpu.CompilerParams(needs_layout_passes=False),
        out_type=(
            jax.ShapeDtypeStruct((BATCH, PW), jnp.float32),
            jax.ShapeDtypeStruct((BATCH,), jnp.float32),
        ),
        scratch_types=[
            pltpu.VMEM((CHUNK,), jnp.int32),
            pltpu.VMEM((CHUNK,), jnp.int32),
            pltpu.VMEM((CHUNK, PW), jnp.float32),
            pltpu.VMEM((CHUNK, PW), jnp.float32),
            pltpu.VMEM((CHUNK,), jnp.float32),
            pltpu.SemaphoreType.DMA,
            pltpu.SemaphoreType.DMA,
            pltpu.SemaphoreType.DMA,
        ],
    )
    def k(u_hbm, i_hbm, p_hbm, s_hbm,
          out_cat, out_s,
          uidx, iidx, urows, irows, svals, sem0, sem1, sem2):
        wid = lax.axis_index("s") * NC + lax.axis_index("c")
        for c in range(NCHUNK):
            rb = pl.multiple_of(wid * ROWS_PER_W + c * CHUNK, CHUNK)
            pltpu.sync_copy(u_hbm.at[pl.ds(rb, CHUNK)], uidx)
            pltpu.sync_copy(i_hbm.at[pl.ds(rb, CHUNK)], iidx)
            cp0 = pltpu.async_copy(s_hbm.at[uidx], svals, sem0)
            cp1 = pltpu.async_copy(p_hbm.at[uidx], urows, sem1)
            cp2 = pltpu.async_copy(p_hbm.at[iidx], irows, sem2)
            cp0.wait()
            pltpu.sync_copy(svals, out_s.at[pl.ds(rb, CHUNK)])
            cp1.wait()
            cp2.wait()

            def merge(r, carry):
                urows[r, pl.ds(DMLP, 16)] = irows[r, pl.ds(DMLP, 16)]
                urows[r, pl.ds(DMLP + 16, 16)] = irows[r, pl.ds(DMLP + 16, 16)]
                return carry

            lax.fori_loop(0, CHUNK, merge, 0)
            pltpu.sync_copy(urows, out_cat.at[pl.ds(rb, CHUNK)])

    return k(userinput, iteminput, p_table, s_table)


def _tc_head(x_cat, mf_part, w1t, b1, w2t, b2, w3t, b3, wpm, bp):
    def body(x_ref, mf_ref, w1_ref, b1_ref, w2_ref, b2_ref,
             w3_ref, b3_ref, wpm_ref, bp_ref, o_ref):
        h = jnp.dot(x_ref[:, 0:2 * DMLP], w1_ref[...],
                    preferred_element_type=jnp.float32)
        h = jnp.maximum(h + b1_ref[...], 0.0)
        h = jnp.maximum(jnp.dot(h, w2_ref[...],
                                preferred_element_type=jnp.float32)
                        + b2_ref[...], 0.0)
        h = jnp.maximum(jnp.dot(h, w3_ref[...],
                                preferred_element_type=jnp.float32)
                        + b3_ref[...], 0.0)
        logit = jnp.sum(h * wpm_ref[...], axis=-1)
        logit = logit + mf_ref[...] + bp_ref[0, 0]
        o_ref[...] = 1.0 / (1.0 + jnp.exp(-logit))

    return pl.pallas_call(
        body,
        out_shape=jax.ShapeDtypeStruct((BATCH,), jnp.float32),
    )(x_cat, mf_part, w1t, b1, w2t, b2, w3t, b3, wpm, bp)


def kernel(userinput, iteminput, mf_user_w, mf_item_w, mlp_user_w,
           mlp_item_w, W1, b1, W2, b2, W3, b3, Wp, bp):
    del mf_item_w
    wp = Wp.reshape(-1)
    p_table, s_table = _tc_build(
        mf_user_w.T, mlp_user_w.T, mlp_item_w.T, wp[:MF].reshape(MF, 1))
    x_cat, mf_part = _sc_gather(userinput, iteminput, p_table, s_table)
    out = _tc_head(
        x_cat, mf_part,
        W1.T, b1.reshape(1, -1),
        W2.T, b2.reshape(1, -1), W3.T, b3.reshape(1, -1),
        wp[MF:].reshape(1, -1), bp.reshape(1, 1))
    return out


# speedup vs baseline: 1.2572x; 1.0167x over previous
import functools

import jax
import jax.numpy as jnp
from jax import lax
from jax.experimental import pallas as pl
from jax.experimental.pallas import tpu as pltpu
from jax.experimental.pallas import tpu_sc as plsc

BATCH = 16384
NUSERS = 100000
MF = 200
DMLP = 32
PW = 128
NC = 2
NS = 16
NW = NC * NS
ROWS_PER_W = BATCH // NW
CHUNK = 128
NCHUNK = ROWS_PER_W // CHUNK
PBLK = 8192


def _tc_build(mf_t, mut, mit, wp_col):
    def body(mft_ref, mut_ref, mit_ref, wp_ref, p_ref, s_ref):
        x = mft_ref[...]
        s_ref[...] = jnp.sum(x * x * wp_ref[...], axis=0)
        p_ref[:, 0:DMLP] = jnp.transpose(mut_ref[...])
        p_ref[:, DMLP:2 * DMLP] = jnp.transpose(mit_ref[...])

    return pl.pallas_call(
        body,
        grid=(pl.cdiv(NUSERS, PBLK),),
        in_specs=[
            pl.BlockSpec((MF, PBLK), lambda i: (0, i)),
            pl.BlockSpec((DMLP, PBLK), lambda i: (0, i)),
            pl.BlockSpec((DMLP, PBLK), lambda i: (0, i)),
            pl.BlockSpec((MF, 1), lambda i: (0, 0)),
        ],
        out_specs=(pl.BlockSpec((PBLK, PW), lambda i: (i, 0)),
                   pl.BlockSpec((PBLK,), lambda i: (i,))),
        out_shape=(jax.ShapeDtypeStruct((NUSERS, PW), jnp.float32),
                   jax.ShapeDtypeStruct((NUSERS,), jnp.float32)),
    )(mf_t, mut, mit, wp_col)


def _sc_gather(userinput, iteminput, p_table, s_table):
    mesh = plsc.VectorSubcoreMesh(core_axis_name="c", subcore_axis_name="s")

    @functools.partial(
        pl.kernel,
        mesh=mesh,
        compiler_params=pltpu.CompilerParams(needs_layout_passes=False),
        out_type=(
            jax.ShapeDtypeStruct((BATCH, PW), jnp.float32),
            jax.ShapeDtypeStruct((BATCH,), jnp.float32),
        ),
        scratch_types=[
            pltpu.VMEM((CHUNK,), jnp.int32),
            pltpu.VMEM((CHUNK,), jnp.int32),
            pltpu.VMEM((CHUNK, PW), jnp.float32),
            pltpu.VMEM((CHUNK, PW), jnp.float32),
            pltpu.VMEM((CHUNK,), jnp.float32),
            pltpu.SemaphoreType.DMA,
            pltpu.SemaphoreType.DMA,
            pltpu.SemaphoreType.DMA,
        ],
    )
    def k(u_hbm, i_hbm, p_hbm, s_hbm,
          out_cat, out_s,
          uidx, iidx, urows, irows, svals, sem0, sem1, sem2):
        wid = lax.axis_index("s") * NC + lax.axis_index("c")
        for c in range(NCHUNK):
            rb = pl.multiple_of(wid * ROWS_PER_W + c * CHUNK, CHUNK)
            pltpu.sync_copy(u_hbm.at[pl.ds(rb, CHUNK)], uidx)
            pltpu.sync_copy(i_hbm.at[pl.ds(rb, CHUNK)], iidx)
            cp0 = pltpu.async_copy(s_hbm.at[uidx], svals, sem0)
            cp1 = pltpu.async_copy(p_hbm.at[uidx], urows, sem1)
            cp2 = pltpu.async_copy(p_hbm.at[iidx], irows, sem2)
            cp0.wait()
            pltpu.sync_copy(svals, out_s.at[pl.ds(rb, CHUNK)])
            cp1.wait()
            cp2.wait()

            def merge(r, carry):
                urows[r, pl.ds(DMLP, 16)] = irows[r, pl.ds(DMLP, 16)]
                urows[r, pl.ds(DMLP + 16, 16)] = irows[r, pl.ds(DMLP + 16, 16)]
                return carry

            lax.fori_loop(0, CHUNK, merge, 0)
            pltpu.sync_copy(urows, out_cat.at[pl.ds(rb, CHUNK)])

    return k(userinput, iteminput, p_table, s_table)


def _tc_head(x_cat, mf_part, w1t, b1, w2t, b2, w3t, b3, wpm, bp):
    def body(x_ref, mf_ref, w1_ref, b1_ref, w2_ref, b2_ref,
             w3_ref, b3_ref, wpm_ref, bp_ref, o_ref):
        h = jnp.dot(x_ref[:, 0:2 * DMLP], w1_ref[...],
                    preferred_element_type=jnp.float32)
        h = jnp.maximum(h + b1_ref[...], 0.0)
        h = jnp.maximum(jnp.dot(h, w2_ref[...],
                                preferred_element_type=jnp.float32)
                        + b2_ref[...], 0.0)
        h = jnp.maximum(jnp.dot(h, w3_ref[...],
                                preferred_element_type=jnp.float32)
                        + b3_ref[...], 0.0)
        logit = jnp.sum(h * wpm_ref[...], axis=-1)
        logit = logit + mf_ref[...] + bp_ref[0, 0]
        o_ref[...] = 1.0 / (1.0 + jnp.exp(-logit))

    return pl.pallas_call(
        body,
        out_shape=jax.ShapeDtypeStruct((BATCH,), jnp.float32),
    )(x_cat, mf_part, w1t, b1, w2t, b2, w3t, b3, wpm, bp)


def kernel(userinput, iteminput, mf_user_w, mf_item_w, mlp_user_w,
           mlp_item_w, W1, b1, W2, b2, W3, b3, Wp, bp):
    del mf_item_w
    wp = Wp.reshape(-1)
    p_table, s_table = _tc_build(
        mf_user_w.T, mlp_user_w.T, mlp_item_w.T, wp[:MF].reshape(MF, 1))
    x_cat, mf_part = _sc_gather(userinput, iteminput, p_table, s_table)
    out = _tc_head(
        x_cat, mf_part,
        W1.T, b1.reshape(1, -1),
        W2.T, b2.reshape(1, -1), W3.T, b3.reshape(1, -1),
        wp[MF:].reshape(1, -1), bp.reshape(1, 1))
    return out


# speedup vs baseline: 1.2757x; 1.0147x over previous
import functools

import jax
import jax.numpy as jnp
from jax import lax
from jax.experimental import pallas as pl
from jax.experimental.pallas import tpu as pltpu
from jax.experimental.pallas import tpu_sc as plsc

BATCH = 16384
NUSERS = 100000
MF = 200
DMLP = 32
PW = 128
NC = 2
NS = 16
NW = NC * NS
ROWS_PER_W = BATCH // NW
CHUNK = 128
NCHUNK = ROWS_PER_W // CHUNK
PBLK = 10240


def _tc_build(mf_t, mut, mit, wp_col):
    def body(mft_ref, mut_ref, mit_ref, wp_ref, p_ref, s_ref):
        x = mft_ref[...]
        s_ref[...] = jnp.sum(x * x * wp_ref[...], axis=0)
        p_ref[:, 0:DMLP] = jnp.transpose(mut_ref[...])
        p_ref[:, DMLP:2 * DMLP] = jnp.transpose(mit_ref[...])

    return pl.pallas_call(
        body,
        grid=(pl.cdiv(NUSERS, PBLK),),
        in_specs=[
            pl.BlockSpec((MF, PBLK), lambda i: (0, i)),
            pl.BlockSpec((DMLP, PBLK), lambda i: (0, i)),
            pl.BlockSpec((DMLP, PBLK), lambda i: (0, i)),
            pl.BlockSpec((MF, 1), lambda i: (0, 0)),
        ],
        out_specs=(pl.BlockSpec((PBLK, PW), lambda i: (i, 0)),
                   pl.BlockSpec((PBLK,), lambda i: (i,))),
        out_shape=(jax.ShapeDtypeStruct((NUSERS, PW), jnp.float32),
                   jax.ShapeDtypeStruct((NUSERS,), jnp.float32)),
    )(mf_t, mut, mit, wp_col)


def _sc_gather(userinput, iteminput, p_table, s_table):
    mesh = plsc.VectorSubcoreMesh(core_axis_name="c", subcore_axis_name="s")

    @functools.partial(
        pl.kernel,
        mesh=mesh,
        compiler_params=pltpu.CompilerParams(needs_layout_passes=False),
        out_type=(
            jax.ShapeDtypeStruct((BATCH, PW), jnp.float32),
            jax.ShapeDtypeStruct((BATCH,), jnp.float32),
        ),
        scratch_types=[
            pltpu.VMEM((CHUNK,), jnp.int32),
            pltpu.VMEM((CHUNK,), jnp.int32),
            pltpu.VMEM((CHUNK, PW), jnp.float32),
            pltpu.VMEM((CHUNK, PW), jnp.float32),
            pltpu.VMEM((CHUNK,), jnp.float32),
            pltpu.SemaphoreType.DMA,
            pltpu.SemaphoreType.DMA,
            pltpu.SemaphoreType.DMA,
        ],
    )
    def k(u_hbm, i_hbm, p_hbm, s_hbm,
          out_cat, out_s,
          uidx, iidx, urows, irows, svals, sem0, sem1, sem2):
        wid = lax.axis_index("s") * NC + lax.axis_index("c")
        for c in range(NCHUNK):
            rb = pl.multiple_of(wid * ROWS_PER_W + c * CHUNK, CHUNK)
            pltpu.sync_copy(u_hbm.at[pl.ds(rb, CHUNK)], uidx)
            pltpu.sync_copy(i_hbm.at[pl.ds(rb, CHUNK)], iidx)
            cp0 = pltpu.async_copy(s_hbm.at[uidx], svals, sem0)
            cp1 = pltpu.async_copy(p_hbm.at[uidx], urows, sem1)
            cp2 = pltpu.async_copy(p_hbm.at[iidx], irows, sem2)
            cp0.wait()
            pltpu.sync_copy(svals, out_s.at[pl.ds(rb, CHUNK)])
            cp1.wait()
            cp2.wait()

            def merge(r, carry):
                urows[r, pl.ds(DMLP, 16)] = irows[r, pl.ds(DMLP, 16)]
                urows[r, pl.ds(DMLP + 16, 16)] = irows[r, pl.ds(DMLP + 16, 16)]
                return carry

            lax.fori_loop(0, CHUNK, merge, 0)
            pltpu.sync_copy(urows, out_cat.at[pl.ds(rb, CHUNK)])

    return k(userinput, iteminput, p_table, s_table)


def _tc_head(x_cat, mf_part, w1t, b1, w2t, b2, w3t, b3, wpm, bp):
    def body(x_ref, mf_ref, w1_ref, b1_ref, w2_ref, b2_ref,
             w3_ref, b3_ref, wpm_ref, bp_ref, o_ref):
        h = jnp.dot(x_ref[:, 0:2 * DMLP], w1_ref[...],
                    preferred_element_type=jnp.float32)
        h = jnp.maximum(h + b1_ref[...], 0.0)
        h = jnp.maximum(jnp.dot(h, w2_ref[...],
                                preferred_element_type=jnp.float32)
                        + b2_ref[...], 0.0)
        h = jnp.maximum(jnp.dot(h, w3_ref[...],
                                preferred_element_type=jnp.float32)
                        + b3_ref[...], 0.0)
        logit = jnp.sum(h * wpm_ref[...], axis=-1)
        logit = logit + mf_ref[...] + bp_ref[0, 0]
        o_ref[...] = 1.0 / (1.0 + jnp.exp(-logit))

    return pl.pallas_call(
        body,
        out_shape=jax.ShapeDtypeStruct((BATCH,), jnp.float32),
    )(x_cat, mf_part, w1t, b1, w2t, b2, w3t, b3, wpm, bp)


def kernel(userinput, iteminput, mf_user_w, mf_item_w, mlp_user_w,
           mlp_item_w, W1, b1, W2, b2, W3, b3, Wp, bp):
    del mf_item_w
    wp = Wp.reshape(-1)
    p_table, s_table = _tc_build(
        mf_user_w.T, mlp_user_w.T, mlp_item_w.T, wp[:MF].reshape(MF, 1))
    x_cat, mf_part = _sc_gather(userinput, iteminput, p_table, s_table)
    out = _tc_head(
        x_cat, mf_part,
        W1.T, b1.reshape(1, -1),
        W2.T, b2.reshape(1, -1), W3.T, b3.reshape(1, -1),
        wp[MF:].reshape(1, -1), bp.reshape(1, 1))
    return out
